# Initial kernel scaffold; baseline (speedup 1.0000x reference)
import jax
import jax.numpy as jnp
from jax.experimental import pallas as pl


def kernel(x, edge_index, W1, as1, ad1, b1, W2, as2, ad2, b2, W3, as3, ad3, b3, Wfc, bfc):
    raise NotImplementedError("write your pallas kernel here")


# speedup vs baseline: 7.1547x; 7.1547x over previous
import functools

import jax
import jax.numpy as jnp
from jax import lax
from jax.experimental import pallas as pl
from jax.experimental.pallas import tpu as pltpu, tpu_sc as plsc

N = 10000
NPAD = 10240
E = 320000
EREAL = E + N
EPAD = 331776
HID = 128

NC, NS, L = 2, 16, 16
NW = NC * NS
EPW = EPAD // NW
CH = 128
NCHUNK = EPW // CH
RPS = NPAD // NS
CB = 32
NQ = 4

_F32 = jnp.float32
_I32 = jnp.int32


def _lanes():
  return lax.iota(_I32, L)


def _splat(v):
  return jnp.full((L,), v, _I32)


def _zeros():
  return jnp.zeros((L,), _F32)


H = 4


@functools.lru_cache(maxsize=None)
def _make_stats_kernel():
  mesh = plsc.VectorSubcoreMesh(
      core_axis_name="c", subcore_axis_name="s", num_cores=NC, num_subcores=NS)

  @functools.partial(
      pl.kernel, mesh=mesh,
      compiler_params=pltpu.CompilerParams(
          use_tc_tiling_on_sc=False, needs_layout_passes=False),
      out_type=[
          jax.ShapeDtypeStruct((H, EPAD), _F32),
          jax.ShapeDtypeStruct((NC, NPAD, 16), _F32),
      ],
      scratch_types=[
          pltpu.VMEM((H, NPAD), _F32),
          pltpu.VMEM((H, NPAD), _F32),
          pltpu.VMEM((CH,), _I32),
          pltpu.VMEM((CH,), _I32),
          pltpu.VMEM((CH, 16), _F32),
          pltpu.VMEM((H, CH), _F32),
          pltpu.VMEM_SHARED((NPAD, 16), _F32),
          pltpu.SemaphoreType.DMA,
      ],
  )
  def stats(as_hbm, ad_hbm, src_hbm, dst_hbm, e_hbm, sc_hbm,
            as_t, ad_t, src_v, dst_v, erows_v, ebuf_v, acc_sh, sem):
    del sem
    c = lax.axis_index("c")
    s = lax.axis_index("s")
    wid = s * NC + c
    base = wid * EPW

    for i in range(CH):
      erows_v[i, :] = _zeros()
    for q in range(RPS // CH):
      pltpu.sync_copy(erows_v, acc_sh.at[pl.ds(s * RPS + q * CH, CH)])
    pltpu.sync_copy(as_hbm, as_t)
    pltpu.sync_copy(ad_hbm, ad_t)
    plsc.subcore_barrier()

    @pl.loop(0, NCHUNK)
    def _chunk(g):
      off = base + g * CH
      pltpu.sync_copy(src_hbm.at[pl.ds(off, CH)], src_v)
      pltpu.sync_copy(dst_hbm.at[pl.ds(off, CH)], dst_v)
      for i in range(CH // L):
        lane = _lanes() + i * L
        s16 = src_v[pl.ds(i * L, L)]
        d16 = dst_v[pl.ds(i * L, L)]
        valid = (off + i * L + _lanes()) < EREAL
        cnt = jnp.where(valid, jnp.full((L,), 1.0, _F32), _zeros())
        plsc.store_scatter(erows_v, [lane, _splat(H)], cnt)
        for h in range(H):
          a_s = plsc.load_gather(as_t, [_splat(h), s16])
          a_d = plsc.load_gather(ad_t, [_splat(h), d16])
          e = a_s + a_d
          e = jnp.maximum(e, 0.2 * e)
          e = jnp.where(valid, e, _zeros())
          plsc.store_scatter(erows_v, [lane, _splat(h)], e)
          ebuf_v[h, pl.ds(i * L, L)] = e
      for h in range(H):
        pltpu.sync_copy(ebuf_v.at[h], e_hbm.at[h, pl.ds(off, CH)])
      pltpu.sync_copy(erows_v, acc_sh.at[dst_v], add=True)

    plsc.subcore_barrier()
    pltpu.sync_copy(acc_sh.at[pl.ds(s * RPS, RPS)],
                    sc_hbm.at[c, pl.ds(s * RPS, RPS)])

  return stats


@functools.lru_cache(maxsize=None)
def _make_agg_kernel():
  mesh = plsc.VectorSubcoreMesh(
      core_axis_name="c", subcore_axis_name="s", num_cores=NC, num_subcores=NS)

  @functools.partial(
      pl.kernel, mesh=mesh,
      compiler_params=pltpu.CompilerParams(
          use_tc_tiling_on_sc=False, needs_layout_passes=False),
      out_type=[
          jax.ShapeDtypeStruct((NC, H, HID // CB, NPAD, CB), _F32),
          jax.ShapeDtypeStruct((NC, NPAD, 16), _F32),
      ],
      scratch_types=[
          pltpu.VMEM((NPAD, H), _F32),
          pltpu.VMEM((16,), _I32),
          pltpu.VMEM((CH, CB), _F32),
          pltpu.VMEM((CH, CB), _F32),
          pltpu.VMEM((CH, 16), _F32),
          pltpu.VMEM((CH,), _I32),
          pltpu.VMEM((CH,), _I32),
          pltpu.VMEM((CH,), _F32),
          pltpu.VMEM_SHARED((NPAD, CB), _F32),
          pltpu.VMEM_SHARED((NPAD, 16), _F32),
          pltpu.SemaphoreType.DMA,
      ],
  )
  def agg(m_hbm, hc_hbm, src_hbm, dst_hbm, e_hbm, hrows_hbm, p_hbm, d_hbm,
          m_t, hc_v, zero_v, rows_v, drows_v,
          src_v, dst_v, e_v, p_acc, d_acc, sem):
    c = lax.axis_index("c")
    s = lax.axis_index("s")
    wid = s * NC + c
    base = wid * EPW
    row0 = s * RPS

    pltpu.sync_copy(m_hbm, m_t)
    pltpu.sync_copy(hc_hbm, hc_v)
    hc = jnp.max(hc_v[...])
    for i in range(CH):
      for q in range(CB // L):
        zero_v[i, pl.ds(q * L, L)] = _zeros()
      drows_v[i, :] = _zeros()
    for q in range(RPS // CH):
      pltpu.sync_copy(drows_v, d_acc.at[pl.ds(row0 + q * CH, CH)])

    @pl.loop(0, H * (HID // CB))
    def _pass(hq):
      h = hq // (HID // CB)
      q = hq % (HID // CB)

      @pl.when(h < hc)
      def _do():
        hsplat = jnp.full((L,), h, _I32)
        for z in range(RPS // CH):
          pltpu.sync_copy(zero_v, p_acc.at[pl.ds(row0 + z * CH, CH)])
        @pl.when(q == 0)
        def _zero_drows():
          for i in range(CH):
            drows_v[i, :] = _zeros()
        plsc.subcore_barrier()

        @pl.loop(0, NCHUNK)
        def _chunk(g):
          off = base + g * CH
          pltpu.sync_copy(src_hbm.at[pl.ds(off, CH)], src_v)
          pltpu.sync_copy(dst_hbm.at[pl.ds(off, CH)], dst_v)
          pltpu.sync_copy(e_hbm.at[h, pl.ds(off, CH)], e_v)
          w_regs = []
          for i in range(CH // L):
            lane = _lanes() + i * L
            d16 = dst_v[pl.ds(i * L, L)]
            m_g = plsc.load_gather(m_t, [d16, hsplat])
            w = jnp.exp(e_v[pl.ds(i * L, L)] - m_g)
            valid = (off + i * L + _lanes()) < EREAL
            w = jnp.where(valid, w, _zeros())
            w_regs.append(w)
            @pl.when(q == 0)
            def _dr():
              plsc.store_scatter(drows_v, [lane, hsplat], w)
          pltpu.async_copy(hrows_hbm.at[h, q].at[src_v], rows_v, sem).wait()
          for i in range(CH // L):
            for jj in range(L):
              wspl = w_regs[i][_splat(jj)]
              j = i * L + jj
              for z in range(CB // L):
                rows_v[j, pl.ds(z * L, L)] = rows_v[j, pl.ds(z * L, L)] * wspl
          pltpu.sync_copy(rows_v, p_acc.at[dst_v], add=True)
          @pl.when(q == 0)
          def _da():
            pltpu.sync_copy(drows_v, d_acc.at[dst_v], add=True)

        plsc.subcore_barrier()
        pltpu.sync_copy(p_acc.at[pl.ds(row0, RPS)],
                        p_hbm.at[c, h, q, pl.ds(row0, RPS)])

    pltpu.sync_copy(d_acc.at[pl.ds(row0, RPS)],
                    d_hbm.at[c, pl.ds(row0, RPS)])

  return agg


@functools.lru_cache(maxsize=None)
def _make_mean():
  def body(sc_ref, m_ref):
    tot = sc_ref[0] + sc_ref[1]
    cnt = jnp.maximum(tot[:, H:H + 1], 1.0)
    m_ref[...] = tot[:, 0:H] / cnt

  return pl.pallas_call(
      body,
      grid=(NPAD // _BR,),
      in_specs=[pl.BlockSpec((NC, _BR, 16), lambda r: (0, r, 0))],
      out_specs=pl.BlockSpec((_BR, H), lambda r: (r, 0)),
      out_shape=jax.ShapeDtypeStruct((NPAD, H), _F32),
  )


_BR = 512


@functools.lru_cache(maxsize=None)
def _make_mm1():
  H = 4

  def body(x_ref, w_ref, as_ref, ad_ref, h_ref, asn_ref, adn_ref):
    hb = jnp.dot(x_ref[...], w_ref[...], preferred_element_type=_F32)
    for h in range(H):
      sl = hb[:, h * HID:(h + 1) * HID]
      h_ref[h] = sl
      asn_ref[h] = jnp.sum(sl * as_ref[h], axis=1)
      adn_ref[h] = jnp.sum(sl * ad_ref[h], axis=1)

  return pl.pallas_call(
      body,
      grid=(NPAD // _BR,),
      in_specs=[
          pl.BlockSpec((_BR, HID), lambda r: (r, 0)),
          pl.BlockSpec((HID, H * HID), lambda r: (0, 0)),
          pl.BlockSpec((H, HID), lambda r: (0, 0)),
          pl.BlockSpec((H, HID), lambda r: (0, 0)),
      ],
      out_specs=[
          pl.BlockSpec((H, _BR, HID), lambda r: (0, r, 0)),
          pl.BlockSpec((H, _BR), lambda r: (0, r)),
          pl.BlockSpec((H, _BR), lambda r: (0, r)),
      ],
      out_shape=[
          jax.ShapeDtypeStruct((H, NPAD, HID), _F32),
          jax.ShapeDtypeStruct((H, NPAD), _F32),
          jax.ShapeDtypeStruct((H, NPAD), _F32),
      ],
  )


@functools.lru_cache(maxsize=None)
def _make_mm_mid(HO):
  HI = 4

  def body(p_ref, d_ref, b_ref, w_ref, as_ref, ad_ref,
           h_ref, asn_ref, adn_ref):
    cols = []
    for g in range(HI):
      num = jnp.concatenate(
          [p_ref[0, g, q] + p_ref[1, g, q] for q in range(NQ)], axis=1)
      den = d_ref[0, :, g:g + 1] + d_ref[1, :, g:g + 1] + 1e-16
      bias = b_ref[0, g * HID:(g + 1) * HID][None, :]
      cols.append(jax.nn.relu(num / den + bias))
    xin = jnp.concatenate(cols, axis=1)
    hb = jnp.dot(xin, w_ref[...], preferred_element_type=_F32)
    for h in range(HO):
      sl = hb[:, h * HID:(h + 1) * HID]
      h_ref[h] = sl
      asn_ref[h] = jnp.sum(sl * as_ref[h], axis=1)
      adn_ref[h] = jnp.sum(sl * ad_ref[h], axis=1)

  return pl.pallas_call(
      body,
      grid=(NPAD // _BR,),
      in_specs=[
          pl.BlockSpec((NC, HI, NQ, _BR, CB), lambda r: (0, 0, 0, r, 0)),
          pl.BlockSpec((NC, _BR, 16), lambda r: (0, r, 0)),
          pl.BlockSpec((1, HI * HID), lambda r: (0, 0)),
          pl.BlockSpec((HI * HID, HO * HID), lambda r: (0, 0)),
          pl.BlockSpec((HO, HID), lambda r: (0, 0)),
          pl.BlockSpec((HO, HID), lambda r: (0, 0)),
      ],
      out_specs=[
          pl.BlockSpec((HO, _BR, HID), lambda r: (0, r, 0)),
          pl.BlockSpec((HO, _BR), lambda r: (0, r)),
          pl.BlockSpec((HO, _BR), lambda r: (0, r)),
      ],
      out_shape=[
          jax.ShapeDtypeStruct((HO, NPAD, HID), _F32),
          jax.ShapeDtypeStruct((HO, NPAD), _F32),
          jax.ShapeDtypeStruct((HO, NPAD), _F32),
      ],
  )


_BRF = 400


@functools.lru_cache(maxsize=None)
def _make_fc(out_ch):
  def body(p_ref, d_ref, b_ref, w_ref, bf_ref, o_ref):
    num = jnp.concatenate(
        [p_ref[0, 0, q] + p_ref[1, 0, q] for q in range(NQ)], axis=1)
    den = d_ref[0, :, 0:1] + d_ref[1, :, 0:1] + 1e-16
    xin = jax.nn.relu(num / den + b_ref[0][None, :])
    o_ref[...] = (jnp.dot(xin, w_ref[...], preferred_element_type=_F32)
                  + bf_ref[0][None, :])

  return pl.pallas_call(
      body,
      grid=(N // _BRF,),
      in_specs=[
          pl.BlockSpec((NC, 1, NQ, _BRF, CB), lambda r: (0, 0, 0, r, 0)),

          pl.BlockSpec((NC, _BRF, 16), lambda r: (0, r, 0)),
          pl.BlockSpec((1, HID), lambda r: (0, 0)),
          pl.BlockSpec((HID, out_ch), lambda r: (0, 0)),
          pl.BlockSpec((1, out_ch), lambda r: (0, 0)),
      ],
      out_specs=pl.BlockSpec((_BRF, out_ch), lambda r: (r, 0)),
      out_shape=jax.ShapeDtypeStruct((N, out_ch), _F32),
  )


def _gat_edge_phase(hc, asn, adn, src, dst, h_table):
  if hc < H:
    asn = jnp.pad(asn, ((0, H - asn.shape[0]), (0, 0)))
    adn = jnp.pad(adn, ((0, H - adn.shape[0]), (0, 0)))
    h_table = jnp.pad(h_table, ((0, H - h_table.shape[0]), (0, 0), (0, 0)))
  e_st, sc_st = _make_stats_kernel()(asn, adn, src, dst)
  m = _make_mean()(sc_st)
  hc_arr = jnp.full((16,), hc, _I32)
  h_blk = jnp.transpose(h_table.reshape(H, NPAD, NQ, CB), (0, 2, 1, 3))
  p_part, d_part = _make_agg_kernel()(m, hc_arr, src, dst, e_st, h_blk)
  return p_part, d_part


def kernel(x, edge_index, W1, as1, ad1, b1, W2, as2, ad2, b2,
           W3, as3, ad3, b3, Wfc, bfc):
  ei = edge_index.astype(_I32)
  loops = jnp.arange(N, dtype=_I32)
  pad = jnp.zeros((EPAD - EREAL,), _I32)
  src = jnp.concatenate([ei[0], loops, pad])
  dst = jnp.concatenate([ei[1], loops, pad])

  xp = jnp.pad(x, ((0, NPAD - N), (0, 0)))

  h1, asn1, adn1 = _make_mm1()(xp, W1, as1, ad1)
  p1, d1 = _gat_edge_phase(4, asn1, adn1, src, dst, h1)

  h2, asn2, adn2 = _make_mm_mid(4)(p1, d1, b1.reshape(1, -1), W2, as2, ad2)
  p2, d2 = _gat_edge_phase(4, asn2, adn2, src, dst, h2)

  h3, asn3, adn3 = _make_mm_mid(1)(p2, d2, b2.reshape(1, -1), W3, as3, ad3)
  p3, d3 = _gat_edge_phase(1, asn3, adn3, src, dst, h3)

  return _make_fc(Wfc.shape[1])(p3, d3, b3.reshape(1, -1), Wfc,
                                bfc.reshape(1, -1))
